# Initial kernel scaffold; baseline (speedup 1.0000x reference)
import jax
import jax.numpy as jnp
from jax.experimental import pallas as pl


def kernel(inputs, table):
    raise NotImplementedError("write your pallas kernel here")


# speedup vs baseline: 1.0685x; 1.0685x over previous
import functools

import jax
import jax.numpy as jnp
from jax import lax
from jax.experimental import pallas as pl
from jax.experimental.pallas import tpu as pltpu
from jax.experimental.pallas import tpu_sc as plsc


def _gather_sc(idx, table, chunk):
    n, = idx.shape
    v, d = table.shape
    info = plsc.get_sparse_core_info()
    nw = info.num_cores * info.num_subcores
    n_per_w = n // nw
    n_chunks = n_per_w // chunk
    mesh = plsc.VectorSubcoreMesh(core_axis_name="c", subcore_axis_name="s")

    @functools.partial(
        pl.kernel,
        mesh=mesh,
        out_type=jax.ShapeDtypeStruct((n, d), jnp.float32),
        scratch_types=[
            pltpu.VMEM((chunk,), jnp.int32),
            pltpu.VMEM((chunk, d), jnp.float32),
            pltpu.SemaphoreType.DMA,
        ],
        compiler_params=pltpu.CompilerParams(use_tc_tiling_on_sc=False),
    )
    def k(idx_hbm, table_hbm, out_hbm, idx_v, rows_v, sem):
        wid = lax.axis_index("s") * info.num_cores + lax.axis_index("c")
        base = wid * n_per_w

        def body(j, carry):
            off = base + j * chunk
            pltpu.sync_copy(idx_hbm.at[pl.ds(off, chunk)], idx_v)
            pltpu.async_copy(table_hbm.at[idx_v], rows_v, sem).wait()
            pltpu.sync_copy(rows_v, out_hbm.at[pl.ds(off, chunk)])
            return carry

        lax.fori_loop(0, n_chunks, body, 0)

    return k(idx, table)


def kernel(inputs, table):
    b, h = inputs.shape
    _, d = table.shape
    idx = inputs.reshape(b * h).astype(jnp.int32)
    out = _gather_sc(idx, table, chunk=1600)
    return out.reshape((-1, d))


# speedup vs baseline: 1.0846x; 1.0150x over previous
import functools

import jax
import jax.numpy as jnp
from jax import lax
from jax.experimental import pallas as pl
from jax.experimental.pallas import tpu as pltpu
from jax.experimental.pallas import tpu_sc as plsc


def _gather_sc(idx, table, chunk):
    n, = idx.shape
    v, d = table.shape
    info = plsc.get_sparse_core_info()
    nw = info.num_cores * info.num_subcores
    n_per_w = n // nw
    n_chunks = n_per_w // chunk
    mesh = plsc.VectorSubcoreMesh(core_axis_name="c", subcore_axis_name="s")

    @functools.partial(
        pl.kernel,
        mesh=mesh,
        out_type=jax.ShapeDtypeStruct((n, d), jnp.float32),
        scratch_types=[
            pltpu.VMEM((2, chunk), jnp.int32),
            pltpu.VMEM((2, chunk, d), jnp.float32),
            pltpu.SemaphoreType.DMA,
            pltpu.SemaphoreType.DMA,
            pltpu.SemaphoreType.DMA,
            pltpu.SemaphoreType.DMA,
            pltpu.SemaphoreType.DMA,
            pltpu.SemaphoreType.DMA,
        ],
        compiler_params=pltpu.CompilerParams(use_tc_tiling_on_sc=False),
    )
    def k(idx_hbm, table_hbm, out_hbm, idx_v, rows_v, si0, si1, sg0, sg1,
          so0, so1):
        wid = lax.axis_index("s") * info.num_cores + lax.axis_index("c")
        base = wid * n_per_w
        si = (si0, si1)
        sg = (sg0, sg1)
        so = (so0, so1)

        idx_d = [None, None]
        gat_d = [None, None]
        out_d = [None, None]

        for b in range(min(2, n_chunks)):
            idx_d[b] = pltpu.async_copy(
                idx_hbm.at[pl.ds(base + b * chunk, chunk)], idx_v.at[b],
                si[b])

        for j in range(n_chunks):
            b = j % 2
            idx_d[b].wait()
            if out_d[b] is not None:
                out_d[b].wait()
            gat_d[b] = pltpu.async_copy(
                table_hbm.at[idx_v.at[b]], rows_v.at[b], sg[b])
            if j >= 1:
                p = (j - 1) % 2
                gat_d[p].wait()
                if j + 1 < n_chunks:
                    idx_d[p] = pltpu.async_copy(
                        idx_hbm.at[pl.ds(base + (j + 1) * chunk, chunk)],
                        idx_v.at[p], si[p])
                out_d[p] = pltpu.async_copy(
                    rows_v.at[p],
                    out_hbm.at[pl.ds(base + (j - 1) * chunk, chunk)], so[p])

        last = (n_chunks - 1) % 2
        gat_d[last].wait()
        out_d[last] = pltpu.async_copy(
            rows_v.at[last],
            out_hbm.at[pl.ds(base + (n_chunks - 1) * chunk, chunk)],
            so[last])
        for b in range(min(2, n_chunks)):
            if out_d[b] is not None:
                out_d[b].wait()

    return k(idx, table)


def kernel(inputs, table):
    b, h = inputs.shape
    _, d = table.shape
    idx = inputs.reshape(b * h).astype(jnp.int32)
    out = _gather_sc(idx, table, chunk=1600)
    return out.reshape((-1, d))
